# baseline (speedup 1.0000x reference)
import functools

import jax
import jax.numpy as jnp
from jax import lax
from jax.experimental import pallas as pl
from jax.experimental.pallas import tpu as pltpu
from jax.experimental.pallas import tpu_sc as plsc

N = 50000
E = 800000
H = 64
K = 128
NCONV = 3
G = 128
CUTOFF = 6.0
GAMMA = 0.5
PI = 3.14159265

NSC = 2
NTILE = 16
LANES = 16
EB = 128
NBLK = E // EB
ROWS_PER_TILE = N // NTILE
ZROWS = 25

EBT = 1024
E2 = ((E + EBT - 1) // EBT) * EBT
NB = 400
assert E2 % EBT == 0 and N % NB == 0

_SC_MESH = dict(core_axis_name="c", subcore_axis_name="s",
                num_cores=NSC, num_subcores=NTILE)


def _sp2(v):
    return jnp.log(jnp.exp(v) + 1.0) - jnp.log(2.0)


def _geom_body(src_hbm, dst_hbm, pos_hbm, ea_hbm, d2_hbm,
               sidx, didx, prs, prd, eab, d2b, isem, gsem):
    c = lax.axis_index("c")
    s = lax.axis_index("s")
    wid = c * NTILE + s
    iot = lax.broadcasted_iota(jnp.int32, (LANES,), 0)
    nw = NSC * NTILE

    def block(blk):
        base = blk * EB
        d1 = pltpu.async_copy(src_hbm.at[pl.ds(base, EB)], sidx, isem)
        d2_ = pltpu.async_copy(dst_hbm.at[pl.ds(base, EB)], didx, isem)
        dea = [pltpu.async_copy(ea_hbm.at[k, pl.ds(base, EB)], eab.at[k], isem)
               for k in range(3)]
        d1.wait(); d2_.wait()
        for d in dea:
            d.wait()
        g1 = pltpu.async_copy(pos_hbm.at[sidx], prs, gsem)
        g2 = pltpu.async_copy(pos_hbm.at[didx], prd, gsem)
        g1.wait(); g2.wait()
        def grp(g, _):
            jb = pl.multiple_of(g * LANES, LANES)
            vx = eab[0, pl.ds(jb, LANES)]
            vy = eab[1, pl.ds(jb, LANES)]
            vz = eab[2, pl.ds(jb, LANES)]
            acc = jnp.zeros((LANES,), jnp.float32)
            for l in range(LANES):
                dv = prd[jb + l, :] - prs[jb + l, :]
                dx = dv[0] + vx[l]
                dy = dv[1] + vy[l]
                dz = dv[2] + vz[l]
                d2s = dx * dx + dy * dy + dz * dz
                acc = jnp.where(iot == l, d2s, acc)
            d2b[pl.ds(jb, LANES)] = acc
            return 0

        lax.fori_loop(0, EB // LANES, grp, 0)
        pltpu.sync_copy(d2b, d2_hbm.at[blk])

    def body(u, _):
        block(u * nw + wid)
        return 0

    lax.fori_loop(0, NBLK // nw, body, 0)

    @pl.when(wid < NBLK - (NBLK // nw) * nw)
    def _():
        block((NBLK // nw) * nw + wid)


def _sc_geom(src, dst, pos16, edge_attr):
    mesh = plsc.VectorSubcoreMesh(**_SC_MESH)
    f = pl.kernel(
        _geom_body,
        out_type=jax.ShapeDtypeStruct((E2 // EB, EB), jnp.float32),
        mesh=mesh,
        scratch_types=[
            pltpu.VMEM((EB,), jnp.int32),
            pltpu.VMEM((EB,), jnp.int32),
            pltpu.VMEM((EB, 16), jnp.float32),
            pltpu.VMEM((EB, 16), jnp.float32),
            pltpu.VMEM((3, EB), jnp.float32),
            pltpu.VMEM((EB,), jnp.float32),
            pltpu.SemaphoreType.DMA,
            pltpu.SemaphoreType.DMA,
        ],
        compiler_params=pltpu.CompilerParams(use_tc_tiling_on_sc=False),
    )
    return f(src, dst, pos16, edge_attr)


NPIPE = 3


def _layer_body(layer, dst_hbm, src_hbm, hf_hbm, w_hbm, agg_hbm,
                didx, didx2, sidx, rows, wbuf, zbuf, aggS, isem, gsem, ssem):
    c = lax.axis_index("c")
    s = lax.axis_index("s")

    z16 = jnp.zeros((LANES,), jnp.float32)
    for j in range(ZROWS):
        zbuf[j, pl.ds(0, LANES)] = z16
        zbuf[j, pl.ds(LANES, LANES)] = z16
    for t in range(ROWS_PER_TILE // ZROWS):
        pltpu.sync_copy(zbuf, aggS.at[pl.ds(s * ROWS_PER_TILE + t * ZROWS, ZROWS)])
    plsc.subcore_barrier()

    coff = c * N
    wrow = 2 * layer + c

    def start_in(i, k):
        b = i * NTILE + s
        base = b * EB
        tc_blk = b // 8
        q = (b % 8) // 2
        r0 = (b % 2) * EB
        wsrc = w_hbm.at[wrow, pl.ds(tc_blk * 256 + r0, EB), pl.ds(q * 32, 32)]
        return (pltpu.async_copy(dst_hbm.at[pl.ds(base, EB)], didx.at[k], isem.at[k]),
                pltpu.async_copy(src_hbm.at[pl.ds(base, EB)], sidx.at[k], isem.at[k]),
                pltpu.async_copy(wsrc, wbuf.at[k], isem.at[k]))

    def adj_and_gather(k):
        for g in range(EB // LANES):
            sl = pl.ds(g * LANES, LANES)
            didx2[k, sl] = didx[k, sl] + coff
        return pltpu.async_copy(hf_hbm.at[didx2.at[k]], rows.at[k], gsem.at[k])

    def mult(k):
        def mrow(g, _):
            for r in range(8):
                j = g * 8 + r
                for h0 in (0, LANES):
                    sl = pl.ds(h0, LANES)
                    rows[k, j, sl] = rows[k, j, sl] * wbuf[k, j, sl]
            return 0
        lax.fori_loop(0, EB // 8, mrow, 0)

    def start_scatter(k):
        return pltpu.async_copy(rows.at[k], aggS.at[sidx.at[k]], ssem.at[k], add=True)

    NMAIN = 130

    def body(q, _):
        i0 = q * NPIPE
        ins = [start_in(i0 + k, k) for k in range(NPIPE)]
        gs = []
        for k in range(NPIPE):
            for d in ins[k]:
                d.wait()
            gs.append(adj_and_gather(k))
        sc = []
        for k in range(NPIPE):
            gs[k].wait()
            mult(k)
            sc.append(start_scatter(k))
        for k in range(NPIPE):
            sc[k].wait()
        return 0

    lax.fori_loop(0, NMAIN, body, 0)

    def block_sync(i):
        ins = start_in(i, 0)
        for d in ins:
            d.wait()
        adj_and_gather(0).wait()
        mult(0)
        start_scatter(0).wait()

    @pl.when(s < NBLK - 390 * NTILE)
    def _():
        block_sync(390)

    plsc.subcore_barrier()
    pltpu.sync_copy(aggS.at[pl.ds(s * ROWS_PER_TILE, ROWS_PER_TILE)],
                    agg_hbm.at[c, pl.ds(s * ROWS_PER_TILE, ROWS_PER_TILE)])


def _sc_layer(layer, dst, src, hf2, wflat):
    mesh = plsc.VectorSubcoreMesh(**_SC_MESH)
    f = pl.kernel(
        functools.partial(_layer_body, layer),
        out_type=jax.ShapeDtypeStruct((NSC, N, 32), jnp.float32),
        mesh=mesh,
        scratch_types=[
            pltpu.VMEM((NPIPE, EB), jnp.int32),
            pltpu.VMEM((NPIPE, EB), jnp.int32),
            pltpu.VMEM((NPIPE, EB), jnp.int32),
            pltpu.VMEM((NPIPE, EB, 32), jnp.float32),
            pltpu.VMEM((NPIPE, EB, 32), jnp.float32),
            pltpu.VMEM((ZROWS, 32), jnp.float32),
            pltpu.VMEM_SHARED((N, 32), jnp.float32),
            pltpu.SemaphoreType.DMA((NPIPE,)),
            pltpu.SemaphoreType.DMA((NPIPE,)),
            pltpu.SemaphoreType.DMA((NPIPE,)),
        ],
        compiler_params=pltpu.CompilerParams(use_tc_tiling_on_sc=False),
    )
    return f(dst, src, hf2, wflat)


def _filters_body(d2_ref, w1_ref, b1_ref, w2_ref, b2_ref, out_ref):
    d2r = d2_ref[...]
    distt = jnp.sqrt(d2r).T
    envt = (1.0 + jnp.cos((PI / CUTOFF) * jnp.sqrt(d2r))).T
    centers = lax.broadcasted_iota(jnp.int32, (1, K), 1).astype(jnp.float32) * (
        CUTOFF / (K - 1))
    ws = [[] for _ in range(NCONV)]
    for grp in range(EBT // EB):
        dcol = distt[:, grp:grp + 1]
        ecol = envt[:, grp:grp + 1]
        rbf = jnp.exp(-GAMMA * (dcol - centers) ** 2)
        for i in range(NCONV):
            w = _sp2(jnp.dot(rbf, w1_ref[i], preferred_element_type=jnp.float32)
                     + b1_ref[i])
            w = _sp2(jnp.dot(w, w2_ref[i], preferred_element_type=jnp.float32)
                     + b2_ref[i])
            ws[i].append(w * ecol)
    Q = EBT // 4
    for i in range(NCONV):
        w = jnp.concatenate(ws[i], axis=0)
        out_ref[2 * i] = jnp.concatenate(
            [w[q * Q:(q + 1) * Q, :32] for q in range(4)], axis=1)
        out_ref[2 * i + 1] = jnp.concatenate(
            [w[q * Q:(q + 1) * Q, 32:] for q in range(4)], axis=1)


def _tc_filters(d2, flt1_W, flt1_b, flt2_W, flt2_b):
    grid = (E2 // EBT,)
    return pl.pallas_call(
        _filters_body,
        grid=grid,
        in_specs=[
            pl.BlockSpec((EBT // EB, EB), lambda j: (j, 0)),
            pl.BlockSpec((NCONV, K, H), lambda j: (0, 0, 0)),
            pl.BlockSpec((NCONV, 1, H), lambda j: (0, 0, 0)),
            pl.BlockSpec((NCONV, H, H), lambda j: (0, 0, 0)),
            pl.BlockSpec((NCONV, 1, H), lambda j: (0, 0, 0)),
        ],
        out_specs=pl.BlockSpec((NCONV * 2, EBT // 4, 128), lambda j: (0, j, 0)),
        out_shape=jax.ShapeDtypeStruct((NCONV * 2, E2 // 4, 128), jnp.float32),
    )(d2, flt1_W, flt1_b, flt2_W, flt2_b)


def _embed_body(x_ref, emb_ref, w_ref, b_ref, h_ref, hf_ref):
    x = x_ref[...]
    h = jnp.where(x == 0, emb_ref[0:1, :],
                  jnp.where(x == 1, emb_ref[1:2, :], emb_ref[2:3, :]))
    hf = jnp.dot(h, w_ref[0], preferred_element_type=jnp.float32) + b_ref[0]
    h_ref[...] = h
    hf_ref[0] = hf[:, :32]
    hf_ref[1] = hf[:, 32:]


def _tc_embed(xcol, emb, lin1_W, lin1_b):
    return pl.pallas_call(
        _embed_body,
        grid=(N // NB,),
        in_specs=[
            pl.BlockSpec((NB, 1), lambda j: (j, 0)),
            pl.BlockSpec((3, H), lambda j: (0, 0)),
            pl.BlockSpec((1, H, H), lambda j: (0, 0, 0)),
            pl.BlockSpec((1, 1, H), lambda j: (0, 0, 0)),
        ],
        out_specs=[
            pl.BlockSpec((NB, H), lambda j: (j, 0)),
            pl.BlockSpec((2, NB, 32), lambda j: (0, j, 0)),
        ],
        out_shape=[
            jax.ShapeDtypeStruct((N, H), jnp.float32),
            jax.ShapeDtypeStruct((2, N, 32), jnp.float32),
        ],
    )(xcol, emb, lin1_W, lin1_b)


def _dense_body(agg_ref, h_ref, w2_ref, b2_ref, w3_ref, b3_ref,
                w1n_ref, b1n_ref, hout_ref, hf_ref):
    agg = jnp.concatenate([agg_ref[0], agg_ref[1]], axis=1)
    t = _sp2(jnp.dot(agg, w2_ref[0], preferred_element_type=jnp.float32)
             + b2_ref[0])
    h2 = jnp.dot(t, w3_ref[0], preferred_element_type=jnp.float32) + b3_ref[0]
    h = h2 + h_ref[...]
    hout_ref[...] = h
    hf = jnp.dot(h, w1n_ref[0], preferred_element_type=jnp.float32) + b1n_ref[0]
    hf_ref[0] = hf[:, :32]
    hf_ref[1] = hf[:, 32:]


def _tc_dense(layer, agg, h, lin2_W, lin2_b, lin3_W, lin3_b, lin1_W, lin1_b):
    i = layer
    return pl.pallas_call(
        _dense_body,
        grid=(N // NB,),
        in_specs=[
            pl.BlockSpec((2, NB, 32), lambda j: (0, j, 0)),
            pl.BlockSpec((NB, H), lambda j: (j, 0)),
            pl.BlockSpec((1, H, H), lambda j: (i, 0, 0)),
            pl.BlockSpec((1, 1, H), lambda j: (i, 0, 0)),
            pl.BlockSpec((1, H, H), lambda j: (i, 0, 0)),
            pl.BlockSpec((1, 1, H), lambda j: (i, 0, 0)),
            pl.BlockSpec((1, H, H), lambda j: (i + 1, 0, 0)),
            pl.BlockSpec((1, 1, H), lambda j: (i + 1, 0, 0)),
        ],
        out_specs=[
            pl.BlockSpec((NB, H), lambda j: (j, 0)),
            pl.BlockSpec((2, NB, 32), lambda j: (0, j, 0)),
        ],
        out_shape=[
            jax.ShapeDtypeStruct((N, H), jnp.float32),
            jax.ShapeDtypeStruct((2, N, 32), jnp.float32),
        ],
    )(agg, h, lin2_W, lin2_b, lin3_W, lin3_b, lin1_W, lin1_b)


def _final_body(agg_ref, h_ref, batch_ref, w2_ref, b2_ref, w3_ref, b3_ref,
                aw1_ref, ab1_ref, aw2_ref, ab2_ref, pool_ref):
    j = pl.program_id(0)
    agg = jnp.concatenate([agg_ref[0], agg_ref[1]], axis=1)
    t = _sp2(jnp.dot(agg, w2_ref[0], preferred_element_type=jnp.float32)
             + b2_ref[0])
    h2 = jnp.dot(t, w3_ref[0], preferred_element_type=jnp.float32) + b3_ref[0]
    h = h2 + h_ref[...]
    t2 = _sp2(jnp.dot(h, aw1_ref[...], preferred_element_type=jnp.float32)
              + ab1_ref[0][None, :])
    o = jnp.dot(t2, aw2_ref[...], preferred_element_type=jnp.float32) + ab2_ref[0][None, :]
    gid = lax.broadcasted_iota(jnp.int32, (1, G), 1)
    onehot = (batch_ref[...] == gid).astype(jnp.float32)
    contrib = jnp.sum(onehot * o, axis=0, keepdims=True)

    @pl.when(j == 0)
    def _():
        pool_ref[...] = jnp.zeros((1, G), jnp.float32)

    pool_ref[...] += contrib


def _tc_final(agg, h, batchcol, lin2_W, lin2_b, lin3_W, lin3_b,
              aw1_W, aw1_b, aw2_W, aw2_b):
    i = NCONV - 1
    return pl.pallas_call(
        _final_body,
        grid=(N // NB,),
        in_specs=[
            pl.BlockSpec((2, NB, 32), lambda j: (0, j, 0)),
            pl.BlockSpec((NB, H), lambda j: (j, 0)),
            pl.BlockSpec((NB, 1), lambda j: (j, 0)),
            pl.BlockSpec((1, H, H), lambda j: (i, 0, 0)),
            pl.BlockSpec((1, 1, H), lambda j: (i, 0, 0)),
            pl.BlockSpec((1, H, H), lambda j: (i, 0, 0)),
            pl.BlockSpec((1, 1, H), lambda j: (i, 0, 0)),
            pl.BlockSpec((H, H // 2), lambda j: (0, 0)),
            pl.BlockSpec((1, H // 2), lambda j: (0, 0)),
            pl.BlockSpec((H // 2, 1), lambda j: (0, 0)),
            pl.BlockSpec((1, 1), lambda j: (0, 0)),
        ],
        out_specs=pl.BlockSpec((1, G), lambda j: (0, 0)),
        out_shape=jax.ShapeDtypeStruct((1, G), jnp.float32),
    )(agg, h, batchcol, lin2_W, lin2_b, lin3_W, lin3_b,
      aw1_W, aw1_b, aw2_W, aw2_b)


def kernel(x, pos, edge_index, edge_attr, batch, emb,
           lin1_W, lin1_b, flt1_W, flt1_b, flt2_W, flt2_b,
           lin2_W, lin2_b, lin3_W, lin3_b, aw1_W, aw1_b, aw2_W, aw2_b):
    src = edge_index[0]
    dst = edge_index[1]
    lin1_b = lin1_b.reshape(NCONV, 1, H)
    flt1_b = flt1_b.reshape(NCONV, 1, H)
    flt2_b = flt2_b.reshape(NCONV, 1, H)
    lin2_b = lin2_b.reshape(NCONV, 1, H)
    lin3_b = lin3_b.reshape(NCONV, 1, H)
    pos16 = jnp.zeros((N, 16), jnp.float32).at[:, :3].set(pos)

    d2 = _sc_geom(src, dst, pos16, edge_attr.T)
    wflat = _tc_filters(d2, flt1_W, flt1_b, flt2_W, flt2_b)

    xcol = x.astype(jnp.int32).reshape(N, 1)
    h, hf = _tc_embed(xcol, emb, lin1_W, lin1_b)

    for i in range(NCONV):
        agg = _sc_layer(i, dst, src, hf.reshape(2 * N, 32), wflat)
        if i < NCONV - 1:
            h, hf = _tc_dense(i, agg, h, lin2_W, lin2_b, lin3_W, lin3_b,
                              lin1_W, lin1_b)
        else:
            pooled = _tc_final(agg, h, batch.astype(jnp.int32).reshape(N, 1),
                               lin2_W, lin2_b, lin3_W, lin3_b,
                               aw1_W, aw1_b.reshape(1, H // 2),
                               aw2_W, aw2_b.reshape(1, 1))
    return pooled.reshape(G, 1)


# speedup vs baseline: 1.0414x; 1.0414x over previous
import functools

import jax
import jax.numpy as jnp
from jax import lax
from jax.experimental import pallas as pl
from jax.experimental.pallas import tpu as pltpu
from jax.experimental.pallas import tpu_sc as plsc

N = 50000
E = 800000
H = 64
K = 128
NCONV = 3
G = 128
CUTOFF = 6.0
GAMMA = 0.5
PI = 3.14159265

NSC = 2
NTILE = 16
LANES = 16
EB = 128
NBLK = E // EB
ROWS_PER_TILE = N // NTILE
ZROWS = 25

EBT = 1024
E2 = ((E + EBT - 1) // EBT) * EBT
NB = 400
assert E2 % EBT == 0 and N % NB == 0

_SC_MESH = dict(core_axis_name="c", subcore_axis_name="s",
                num_cores=NSC, num_subcores=NTILE)


def _sp2(v):
    return jnp.log(jnp.exp(v) + 1.0) - jnp.log(2.0)


def _geom_body(src_hbm, dst_hbm, pos_hbm, ea_hbm, d2_hbm,
               sidx, didx, prs, prd, eab, d2b, isem, gsem, osem):
    c = lax.axis_index("c")
    s = lax.axis_index("s")
    wid = c * NTILE + s
    iot = lax.broadcasted_iota(jnp.int32, (LANES,), 0)
    nw = NSC * NTILE

    def start_in(blk, k):
        base = blk * EB
        return (pltpu.async_copy(src_hbm.at[pl.ds(base, EB)], sidx.at[k], isem.at[k]),
                pltpu.async_copy(dst_hbm.at[pl.ds(base, EB)], didx.at[k], isem.at[k])) + tuple(
            pltpu.async_copy(ea_hbm.at[kk, pl.ds(base, EB)], eab.at[k, kk], isem.at[k])
            for kk in range(3))

    def start_gather(k):
        return (pltpu.async_copy(pos_hbm.at[sidx.at[k]], prs.at[k], gsem.at[k]),
                pltpu.async_copy(pos_hbm.at[didx.at[k]], prd.at[k], gsem.at[k]))

    def compute(k):
        def grp(g, _):
            jb = pl.multiple_of(g * LANES, LANES)
            vx = eab[k, 0, pl.ds(jb, LANES)]
            vy = eab[k, 1, pl.ds(jb, LANES)]
            vz = eab[k, 2, pl.ds(jb, LANES)]
            acc = jnp.zeros((LANES,), jnp.float32)
            for l in range(LANES):
                dv = prd[k, jb + l, :] - prs[k, jb + l, :]
                dx = dv[0] + vx[l]
                dy = dv[1] + vy[l]
                dz = dv[2] + vz[l]
                d2s = dx * dx + dy * dy + dz * dz
                acc = jnp.where(iot == l, d2s, acc)
            d2b[k, pl.ds(jb, LANES)] = acc
            return 0

        lax.fori_loop(0, EB // LANES, grp, 0)

    def start_out(blk, k):
        return pltpu.async_copy(d2b.at[k], d2_hbm.at[blk], osem.at[k])

    def body(u, _):
        b0 = (2 * u) * nw + wid
        b1 = (2 * u + 1) * nw + wid
        in0 = start_in(b0, 0)
        in1 = start_in(b1, 1)
        for d in in0:
            d.wait()
        g0 = start_gather(0)
        for d in in1:
            d.wait()
        g1 = start_gather(1)
        for d in g0:
            d.wait()
        compute(0)
        o0 = start_out(b0, 0)
        for d in g1:
            d.wait()
        compute(1)
        o1 = start_out(b1, 1)
        o0.wait()
        o1.wait()
        return 0

    lax.fori_loop(0, NBLK // nw // 2, body, 0)

    def block_sync(blk):
        for d in start_in(blk, 0):
            d.wait()
        for d in start_gather(0):
            d.wait()
        compute(0)
        start_out(blk, 0).wait()

    nfull = (NBLK // nw // 2) * 2
    block_sync(nfull * nw + wid)

    @pl.when(wid < NBLK - (NBLK // nw) * nw)
    def _():
        block_sync((NBLK // nw) * nw + wid)


def _sc_geom(src, dst, pos16, edge_attr):
    mesh = plsc.VectorSubcoreMesh(**_SC_MESH)
    f = pl.kernel(
        _geom_body,
        out_type=jax.ShapeDtypeStruct((E2 // EB, EB), jnp.float32),
        mesh=mesh,
        scratch_types=[
            pltpu.VMEM((2, EB), jnp.int32),
            pltpu.VMEM((2, EB), jnp.int32),
            pltpu.VMEM((2, EB, 16), jnp.float32),
            pltpu.VMEM((2, EB, 16), jnp.float32),
            pltpu.VMEM((2, 3, EB), jnp.float32),
            pltpu.VMEM((2, EB), jnp.float32),
            pltpu.SemaphoreType.DMA((2,)),
            pltpu.SemaphoreType.DMA((2,)),
            pltpu.SemaphoreType.DMA((2,)),
        ],
        compiler_params=pltpu.CompilerParams(use_tc_tiling_on_sc=False),
    )
    return f(src, dst, pos16, edge_attr)


NPIPE = 3


def _layer_body(layer, dst_hbm, src_hbm, hf_hbm, w_hbm, agg_hbm,
                didx, didx2, sidx, rows, wbuf, zbuf, aggS, isem, gsem, ssem):
    c = lax.axis_index("c")
    s = lax.axis_index("s")

    z16 = jnp.zeros((LANES,), jnp.float32)
    for j in range(ZROWS):
        zbuf[j, pl.ds(0, LANES)] = z16
        zbuf[j, pl.ds(LANES, LANES)] = z16
    for t in range(ROWS_PER_TILE // ZROWS):
        pltpu.sync_copy(zbuf, aggS.at[pl.ds(s * ROWS_PER_TILE + t * ZROWS, ZROWS)])
    plsc.subcore_barrier()

    coff = c * N
    wrow = 2 * layer + c

    def start_in(i, k):
        b = i * NTILE + s
        base = b * EB
        tc_blk = b // 8
        q = (b % 8) // 2
        r0 = (b % 2) * EB
        wsrc = w_hbm.at[wrow, pl.ds(tc_blk * 256 + r0, EB), pl.ds(q * 32, 32)]
        return (pltpu.async_copy(dst_hbm.at[pl.ds(base, EB)], didx.at[k], isem.at[k]),
                pltpu.async_copy(src_hbm.at[pl.ds(base, EB)], sidx.at[k], isem.at[k]),
                pltpu.async_copy(wsrc, wbuf.at[k], isem.at[k]))

    def adj_and_gather(k):
        for g in range(EB // LANES):
            sl = pl.ds(g * LANES, LANES)
            didx2[k, sl] = didx[k, sl] + coff
        return pltpu.async_copy(hf_hbm.at[didx2.at[k]], rows.at[k], gsem.at[k])

    def mult(k):
        def mrow(g, _):
            for r in range(8):
                j = g * 8 + r
                for h0 in (0, LANES):
                    sl = pl.ds(h0, LANES)
                    rows[k, j, sl] = rows[k, j, sl] * wbuf[k, j, sl]
            return 0
        lax.fori_loop(0, EB // 8, mrow, 0)

    def start_scatter(k):
        return pltpu.async_copy(rows.at[k], aggS.at[sidx.at[k]], ssem.at[k], add=True)

    NMAIN = 130

    def body(q, _):
        i0 = q * NPIPE
        ins = [start_in(i0 + k, k) for k in range(NPIPE)]
        gs = []
        for k in range(NPIPE):
            for d in ins[k]:
                d.wait()
            gs.append(adj_and_gather(k))
        sc = []
        for k in range(NPIPE):
            gs[k].wait()
            mult(k)
            sc.append(start_scatter(k))
        for k in range(NPIPE):
            sc[k].wait()
        return 0

    lax.fori_loop(0, NMAIN, body, 0)

    def block_sync(i):
        ins = start_in(i, 0)
        for d in ins:
            d.wait()
        adj_and_gather(0).wait()
        mult(0)
        start_scatter(0).wait()

    @pl.when(s < NBLK - 390 * NTILE)
    def _():
        block_sync(390)

    plsc.subcore_barrier()
    pltpu.sync_copy(aggS.at[pl.ds(s * ROWS_PER_TILE, ROWS_PER_TILE)],
                    agg_hbm.at[c, pl.ds(s * ROWS_PER_TILE, ROWS_PER_TILE)])


def _sc_layer(layer, dst, src, hf2, wflat):
    mesh = plsc.VectorSubcoreMesh(**_SC_MESH)
    f = pl.kernel(
        functools.partial(_layer_body, layer),
        out_type=jax.ShapeDtypeStruct((NSC, N, 32), jnp.float32),
        mesh=mesh,
        scratch_types=[
            pltpu.VMEM((NPIPE, EB), jnp.int32),
            pltpu.VMEM((NPIPE, EB), jnp.int32),
            pltpu.VMEM((NPIPE, EB), jnp.int32),
            pltpu.VMEM((NPIPE, EB, 32), jnp.float32),
            pltpu.VMEM((NPIPE, EB, 32), jnp.float32),
            pltpu.VMEM((ZROWS, 32), jnp.float32),
            pltpu.VMEM_SHARED((N, 32), jnp.float32),
            pltpu.SemaphoreType.DMA((NPIPE,)),
            pltpu.SemaphoreType.DMA((NPIPE,)),
            pltpu.SemaphoreType.DMA((NPIPE,)),
        ],
        compiler_params=pltpu.CompilerParams(use_tc_tiling_on_sc=False),
    )
    return f(dst, src, hf2, wflat)


def _filters_body(d2_ref, w1_ref, b1_ref, w2_ref, b2_ref, out_ref):
    d2r = d2_ref[...]
    distt = jnp.sqrt(d2r).T
    envt = (1.0 + jnp.cos((PI / CUTOFF) * jnp.sqrt(d2r))).T
    centers = lax.broadcasted_iota(jnp.int32, (1, K), 1).astype(jnp.float32) * (
        CUTOFF / (K - 1))
    ws = [[] for _ in range(NCONV)]
    for grp in range(EBT // EB):
        dcol = distt[:, grp:grp + 1]
        ecol = envt[:, grp:grp + 1]
        rbf = jnp.exp(-GAMMA * (dcol - centers) ** 2)
        for i in range(NCONV):
            w = _sp2(jnp.dot(rbf, w1_ref[i], preferred_element_type=jnp.float32)
                     + b1_ref[i])
            w = _sp2(jnp.dot(w, w2_ref[i], preferred_element_type=jnp.float32)
                     + b2_ref[i])
            ws[i].append(w * ecol)
    Q = EBT // 4
    for i in range(NCONV):
        w = jnp.concatenate(ws[i], axis=0)
        out_ref[2 * i] = jnp.concatenate(
            [w[q * Q:(q + 1) * Q, :32] for q in range(4)], axis=1)
        out_ref[2 * i + 1] = jnp.concatenate(
            [w[q * Q:(q + 1) * Q, 32:] for q in range(4)], axis=1)


def _tc_filters(d2, flt1_W, flt1_b, flt2_W, flt2_b):
    grid = (E2 // EBT,)
    return pl.pallas_call(
        _filters_body,
        grid=grid,
        in_specs=[
            pl.BlockSpec((EBT // EB, EB), lambda j: (j, 0)),
            pl.BlockSpec((NCONV, K, H), lambda j: (0, 0, 0)),
            pl.BlockSpec((NCONV, 1, H), lambda j: (0, 0, 0)),
            pl.BlockSpec((NCONV, H, H), lambda j: (0, 0, 0)),
            pl.BlockSpec((NCONV, 1, H), lambda j: (0, 0, 0)),
        ],
        out_specs=pl.BlockSpec((NCONV * 2, EBT // 4, 128), lambda j: (0, j, 0)),
        out_shape=jax.ShapeDtypeStruct((NCONV * 2, E2 // 4, 128), jnp.float32),
    )(d2, flt1_W, flt1_b, flt2_W, flt2_b)


def _embed_body(x_ref, emb_ref, w_ref, b_ref, h_ref, hf_ref):
    x = x_ref[...]
    h = jnp.where(x == 0, emb_ref[0:1, :],
                  jnp.where(x == 1, emb_ref[1:2, :], emb_ref[2:3, :]))
    hf = jnp.dot(h, w_ref[0], preferred_element_type=jnp.float32) + b_ref[0]
    h_ref[...] = h
    hf_ref[0] = hf[:, :32]
    hf_ref[1] = hf[:, 32:]


def _tc_embed(xcol, emb, lin1_W, lin1_b):
    return pl.pallas_call(
        _embed_body,
        grid=(N // NB,),
        in_specs=[
            pl.BlockSpec((NB, 1), lambda j: (j, 0)),
            pl.BlockSpec((3, H), lambda j: (0, 0)),
            pl.BlockSpec((1, H, H), lambda j: (0, 0, 0)),
            pl.BlockSpec((1, 1, H), lambda j: (0, 0, 0)),
        ],
        out_specs=[
            pl.BlockSpec((NB, H), lambda j: (j, 0)),
            pl.BlockSpec((2, NB, 32), lambda j: (0, j, 0)),
        ],
        out_shape=[
            jax.ShapeDtypeStruct((N, H), jnp.float32),
            jax.ShapeDtypeStruct((2, N, 32), jnp.float32),
        ],
    )(xcol, emb, lin1_W, lin1_b)


def _dense_body(agg_ref, h_ref, w2_ref, b2_ref, w3_ref, b3_ref,
                w1n_ref, b1n_ref, hout_ref, hf_ref):
    agg = jnp.concatenate([agg_ref[0], agg_ref[1]], axis=1)
    t = _sp2(jnp.dot(agg, w2_ref[0], preferred_element_type=jnp.float32)
             + b2_ref[0])
    h2 = jnp.dot(t, w3_ref[0], preferred_element_type=jnp.float32) + b3_ref[0]
    h = h2 + h_ref[...]
    hout_ref[...] = h
    hf = jnp.dot(h, w1n_ref[0], preferred_element_type=jnp.float32) + b1n_ref[0]
    hf_ref[0] = hf[:, :32]
    hf_ref[1] = hf[:, 32:]


def _tc_dense(layer, agg, h, lin2_W, lin2_b, lin3_W, lin3_b, lin1_W, lin1_b):
    i = layer
    return pl.pallas_call(
        _dense_body,
        grid=(N // NB,),
        in_specs=[
            pl.BlockSpec((2, NB, 32), lambda j: (0, j, 0)),
            pl.BlockSpec((NB, H), lambda j: (j, 0)),
            pl.BlockSpec((1, H, H), lambda j: (i, 0, 0)),
            pl.BlockSpec((1, 1, H), lambda j: (i, 0, 0)),
            pl.BlockSpec((1, H, H), lambda j: (i, 0, 0)),
            pl.BlockSpec((1, 1, H), lambda j: (i, 0, 0)),
            pl.BlockSpec((1, H, H), lambda j: (i + 1, 0, 0)),
            pl.BlockSpec((1, 1, H), lambda j: (i + 1, 0, 0)),
        ],
        out_specs=[
            pl.BlockSpec((NB, H), lambda j: (j, 0)),
            pl.BlockSpec((2, NB, 32), lambda j: (0, j, 0)),
        ],
        out_shape=[
            jax.ShapeDtypeStruct((N, H), jnp.float32),
            jax.ShapeDtypeStruct((2, N, 32), jnp.float32),
        ],
    )(agg, h, lin2_W, lin2_b, lin3_W, lin3_b, lin1_W, lin1_b)


def _final_body(agg_ref, h_ref, batch_ref, w2_ref, b2_ref, w3_ref, b3_ref,
                aw1_ref, ab1_ref, aw2_ref, ab2_ref, pool_ref):
    j = pl.program_id(0)
    agg = jnp.concatenate([agg_ref[0], agg_ref[1]], axis=1)
    t = _sp2(jnp.dot(agg, w2_ref[0], preferred_element_type=jnp.float32)
             + b2_ref[0])
    h2 = jnp.dot(t, w3_ref[0], preferred_element_type=jnp.float32) + b3_ref[0]
    h = h2 + h_ref[...]
    t2 = _sp2(jnp.dot(h, aw1_ref[...], preferred_element_type=jnp.float32)
              + ab1_ref[0][None, :])
    o = jnp.dot(t2, aw2_ref[...], preferred_element_type=jnp.float32) + ab2_ref[0][None, :]
    gid = lax.broadcasted_iota(jnp.int32, (1, G), 1)
    onehot = (batch_ref[...] == gid).astype(jnp.float32)
    contrib = jnp.sum(onehot * o, axis=0, keepdims=True)

    @pl.when(j == 0)
    def _():
        pool_ref[...] = jnp.zeros((1, G), jnp.float32)

    pool_ref[...] += contrib


def _tc_final(agg, h, batchcol, lin2_W, lin2_b, lin3_W, lin3_b,
              aw1_W, aw1_b, aw2_W, aw2_b):
    i = NCONV - 1
    return pl.pallas_call(
        _final_body,
        grid=(N // NB,),
        in_specs=[
            pl.BlockSpec((2, NB, 32), lambda j: (0, j, 0)),
            pl.BlockSpec((NB, H), lambda j: (j, 0)),
            pl.BlockSpec((NB, 1), lambda j: (j, 0)),
            pl.BlockSpec((1, H, H), lambda j: (i, 0, 0)),
            pl.BlockSpec((1, 1, H), lambda j: (i, 0, 0)),
            pl.BlockSpec((1, H, H), lambda j: (i, 0, 0)),
            pl.BlockSpec((1, 1, H), lambda j: (i, 0, 0)),
            pl.BlockSpec((H, H // 2), lambda j: (0, 0)),
            pl.BlockSpec((1, H // 2), lambda j: (0, 0)),
            pl.BlockSpec((H // 2, 1), lambda j: (0, 0)),
            pl.BlockSpec((1, 1), lambda j: (0, 0)),
        ],
        out_specs=pl.BlockSpec((1, G), lambda j: (0, 0)),
        out_shape=jax.ShapeDtypeStruct((1, G), jnp.float32),
    )(agg, h, batchcol, lin2_W, lin2_b, lin3_W, lin3_b,
      aw1_W, aw1_b, aw2_W, aw2_b)


def kernel(x, pos, edge_index, edge_attr, batch, emb,
           lin1_W, lin1_b, flt1_W, flt1_b, flt2_W, flt2_b,
           lin2_W, lin2_b, lin3_W, lin3_b, aw1_W, aw1_b, aw2_W, aw2_b):
    src = edge_index[0]
    dst = edge_index[1]
    lin1_b = lin1_b.reshape(NCONV, 1, H)
    flt1_b = flt1_b.reshape(NCONV, 1, H)
    flt2_b = flt2_b.reshape(NCONV, 1, H)
    lin2_b = lin2_b.reshape(NCONV, 1, H)
    lin3_b = lin3_b.reshape(NCONV, 1, H)
    pos16 = jnp.zeros((N, 16), jnp.float32).at[:, :3].set(pos)

    d2 = _sc_geom(src, dst, pos16, edge_attr.T)
    wflat = _tc_filters(d2, flt1_W, flt1_b, flt2_W, flt2_b)

    xcol = x.astype(jnp.int32).reshape(N, 1)
    h, hf = _tc_embed(xcol, emb, lin1_W, lin1_b)

    for i in range(NCONV):
        agg = _sc_layer(i, dst, src, hf.reshape(2 * N, 32), wflat)
        if i < NCONV - 1:
            h, hf = _tc_dense(i, agg, h, lin2_W, lin2_b, lin3_W, lin3_b,
                              lin1_W, lin1_b)
        else:
            pooled = _tc_final(agg, h, batch.astype(jnp.int32).reshape(N, 1),
                               lin2_W, lin2_b, lin3_W, lin3_b,
                               aw1_W, aw1_b.reshape(1, H // 2),
                               aw2_W, aw2_b.reshape(1, 1))
    return pooled.reshape(G, 1)
